# baseline (speedup 1.0000x reference)
import functools

import jax
import jax.numpy as jnp
from jax import lax
from jax.experimental import pallas as pl
from jax.experimental.pallas import tpu as pltpu
from jax.experimental.pallas import tpu_sc as plsc

_NC = 2
_NS = 16
_NW = _NC * _NS


@functools.partial(jax.jit, static_argnums=(4, 5))
def _gather_sc(uidx, iidx, user_table, item_table, B, D):
    b_per_w = B // _NW
    mesh = plsc.VectorSubcoreMesh(core_axis_name="c", subcore_axis_name="s")

    @functools.partial(
        pl.kernel,
        mesh=mesh,
        compiler_params=pltpu.CompilerParams(use_tc_tiling_on_sc=False),
        out_type=[
            jax.ShapeDtypeStruct((B, D), jnp.float32),
            jax.ShapeDtypeStruct((B, D), jnp.float32),
        ],
        scratch_types=[
            pltpu.VMEM((b_per_w,), jnp.int32),
            pltpu.VMEM((b_per_w,), jnp.int32),
            pltpu.VMEM((b_per_w, D), jnp.float32),
            pltpu.VMEM((b_per_w, D), jnp.float32),
            pltpu.SemaphoreType.DMA,
            pltpu.SemaphoreType.DMA,
        ],
    )
    def k(uidx_hbm, iidx_hbm, ut_hbm, it_hbm, ue_hbm, ie_hbm,
          uix, iix, urows, irows, sem_u, sem_i):
        wid = lax.axis_index("s") * _NC + lax.axis_index("c")
        base = wid * b_per_w

        pltpu.sync_copy(uidx_hbm.at[pl.ds(base, b_per_w)], uix)
        pltpu.sync_copy(iidx_hbm.at[pl.ds(base, b_per_w)], iix)

        cp_u = pltpu.async_copy(ut_hbm.at[uix], urows, sem_u)
        cp_i = pltpu.async_copy(it_hbm.at[iix], irows, sem_i)
        cp_u.wait()
        cp_i.wait()

        pltpu.sync_copy(urows, ue_hbm.at[pl.ds(base, b_per_w)])
        pltpu.sync_copy(irows, ie_hbm.at[pl.ds(base, b_per_w)])

    return k(uidx, iidx, user_table, item_table)


def _dot_body(u_ref, i_ref, b_ref, o_ref):
    o_ref[...] = (
        jnp.sum(u_ref[...] * i_ref[...], axis=1, keepdims=True) + b_ref[0]
    )


@functools.partial(jax.jit, static_argnums=(3, 4, 5))
def _dot_tc(ue, ie, bias, B, D, blk):
    return pl.pallas_call(
        _dot_body,
        grid=(B // blk,),
        in_specs=[
            pl.BlockSpec((blk, D), lambda i: (i, 0)),
            pl.BlockSpec((blk, D), lambda i: (i, 0)),
            pl.BlockSpec(memory_space=pltpu.SMEM),
        ],
        out_specs=pl.BlockSpec((blk, 1), lambda i: (i, 0)),
        out_shape=jax.ShapeDtypeStruct((B, 1), jnp.float32),
    )(ue, ie, bias)


def kernel(x, user_table, item_table, bias):
    B = x.shape[0]
    D = user_table.shape[1]
    uidx = x[:, 0]
    iidx = x[:, 1]
    ue, ie = _gather_sc(uidx, iidx, user_table, item_table, B, D)
    out = _dot_tc(ue, ie, bias, B, D, 2048)
    return out, ue, ie
